# baseline (speedup 1.0000x reference)
import functools

import jax
import jax.numpy as jnp
from jax import lax
from jax.experimental import pallas as pl
from jax.experimental.pallas import tpu as pltpu
from jax.experimental.pallas import tpu_sc as plsc

_NC = 2
_NS = 16
_NW = _NC * _NS
_L = 16


def _rne_bf16_bits(x):
    b = lax.bitcast_convert_type(x, jnp.int32)
    r = b + 0x7FFF + (lax.shift_right_logical(b, 16) & 1)
    return lax.shift_right_logical(r, 16)


def _encode_body(ax_ref, px_ref, pae_ref, pao_ref, ppe_ref, ppo_ref, c_ref,
                 oa_ref, op_ref):
    ae = jnp.dot(ax_ref[...], pae_ref[...], preferred_element_type=jnp.float32)
    ao = jnp.dot(ax_ref[...], pao_ref[...], preferred_element_type=jnp.float32)
    oa_ref[...] = _rne_bf16_bits(ae) | (_rne_bf16_bits(ao) << 16)
    cc = c_ref[0]
    pe = jnp.dot(px_ref[...], ppe_ref[...], preferred_element_type=jnp.float32)
    po = jnp.dot(px_ref[...], ppo_ref[...], preferred_element_type=jnp.float32)
    op_ref[...] = _rne_bf16_bits(pe + cc) | (_rne_bf16_bits(po + cc) << 16)


def _encode(ax2, px2, pae, pao, ppe, ppo, const):
    ra = ax2.shape[0]
    rp = px2.shape[0]
    return pl.pallas_call(
        _encode_body,
        out_shape=[
            jax.ShapeDtypeStruct((ra, 128), jnp.int32),
            jax.ShapeDtypeStruct((rp, 128), jnp.int32),
        ],
        in_specs=[
            pl.BlockSpec(memory_space=pltpu.VMEM),
            pl.BlockSpec(memory_space=pltpu.VMEM),
            pl.BlockSpec(memory_space=pltpu.VMEM),
            pl.BlockSpec(memory_space=pltpu.VMEM),
            pl.BlockSpec(memory_space=pltpu.VMEM),
            pl.BlockSpec(memory_space=pltpu.VMEM),
            pl.BlockSpec(memory_space=pltpu.SMEM),
        ],
    )(ax2, px2, pae, pao, ppe, ppo, const)


_CROWS = 32
_KE = _CROWS * 128


def _make_edge_kernel(rows_total, ta_rows, tp_rows):
    rows8 = rows_total // 8
    base8 = rows8 // _NW
    extra8 = rows8 - base8 * _NW
    chunks = -(-(base8 + 1) * 8 // _CROWS)
    steps = _KE // _L
    mesh = plsc.VectorSubcoreMesh(
        core_axis_name="c", subcore_axis_name="s",
        num_cores=_NC, num_subcores=_NS)

    @functools.partial(
        pl.kernel,
        out_type=jax.ShapeDtypeStruct((rows_total, 128), jnp.float32),
        mesh=mesh,
        compiler_params=pltpu.CompilerParams(needs_layout_passes=False),
        scratch_types=[
            pltpu.VMEM((ta_rows, 128), jnp.int32),
            pltpu.VMEM((tp_rows, 128), jnp.int32),
            pltpu.VMEM((_CROWS, 128), jnp.int32),
            pltpu.VMEM((_CROWS, 128), jnp.int32),
            pltpu.VMEM((_CROWS, 128), jnp.float32),
        ],
    )
    def edge_kernel(sa_hbm, sp_hbm, src_hbm, dst_hbm, out_hbm,
                    sa_v, sp_v, src_v, dst_v, out_v):
        wid = lax.axis_index("s") * _NC + lax.axis_index("c")
        row_lo = (wid * base8 + jnp.minimum(wid, extra8)) * 8
        n_rows = (base8 + jnp.where(wid < extra8, 1, 0)) * 8
        last_off = row_lo + n_rows - _CROWS
        pltpu.sync_copy(sa_hbm, sa_v)
        pltpu.sync_copy(sp_hbm, sp_v)

        def chunk(c, carry):
            off = pl.multiple_of(
                jnp.minimum(row_lo + c * _CROWS, last_off), 8)
            pltpu.sync_copy(src_hbm.at[pl.ds(off, _CROWS), :], src_v)
            pltpu.sync_copy(dst_hbm.at[pl.ds(off, _CROWS), :], dst_v)

            def inner(i, carry2):
                r = i >> 3
                cg = (i & 7) * _L
                sl = pl.ds(cg, _L)
                s = src_v[r, sl]
                d = dst_v[r, sl]
                ws = plsc.load_gather(sa_v, [s >> 8, (s >> 1) & 127])
                wd = plsc.load_gather(sp_v, [d >> 8, (d >> 1) & 127])
                fs = plsc.bitcast(
                    lax.shift_right_logical(ws, (s & 1) << 4) << 16,
                    jnp.float32)
                fd = plsc.bitcast(
                    lax.shift_right_logical(wd, (d & 1) << 4) << 16,
                    jnp.float32)
                out_v[r, sl] = fs + fd
                return carry2

            lax.fori_loop(0, steps, inner, 0, unroll=2)
            pltpu.sync_copy(out_v, out_hbm.at[pl.ds(off, _CROWS), :])
            return carry

        lax.fori_loop(0, chunks, chunk, 0)

    return edge_kernel


def _pattern(v, d):
    l = jnp.arange(256 * d)
    c = jnp.arange(128)
    vt = jnp.tile(v, 256)
    node = l[:, None] // d
    pe = jnp.where(node == 2 * c[None, :], vt[:, None], 0.0)
    po = jnp.where(node == 2 * c[None, :] + 1, vt[:, None], 0.0)
    return pe.astype(jnp.float32), po.astype(jnp.float32)


def _pad_nodes(x, n_pad):
    n = x.shape[0]
    if n_pad != n:
        x = jnp.concatenate(
            [x, jnp.zeros((n_pad - n, x.shape[1]), x.dtype)], axis=0)
    return x


def kernel(author_x, paper_x, src_index, dst_index,
           W_author, b_author, W_paper, b_paper, W_scorer, b_scorer):
    d = author_x.shape[1]
    na = author_x.shape[0]
    npp = paper_x.shape[0]
    e = src_index.shape[0]
    npn = 1024 // d

    w1 = W_scorer[0, :d]
    w2 = W_scorer[0, d:]
    v_a = W_author.T @ w1
    v_p = W_paper.T @ w2
    const = (b_scorer[0] + b_author @ w1 + b_paper @ w2).reshape(1)

    pae, pao = _pattern(v_a, d)
    ppe, ppo = _pattern(v_p, d)

    na_pad = -(-na // npn) * npn
    np_pad = -(-npp // npn) * npn
    ax2 = _pad_nodes(author_x, na_pad).reshape(na_pad // npn, npn * d)
    px2 = _pad_nodes(paper_x, np_pad).reshape(np_pad // npn, npn * d)

    sa_pk, sp_pk = _encode(ax2, px2, pae, pao, ppe, ppo, const)

    src = src_index.astype(jnp.int32)
    dst = dst_index.astype(jnp.int32)
    min_e = _KE * _NW
    e_pad = -(-max(e, min_e) // 1024) * 1024
    if e_pad != e:
        pad = e_pad - e
        src = jnp.concatenate([src, jnp.zeros((pad,), jnp.int32)])
        dst = jnp.concatenate([dst, jnp.zeros((pad,), jnp.int32)])
    rows = e_pad // 128

    edge_kernel = _make_edge_kernel(rows, sa_pk.shape[0], sp_pk.shape[0])
    out2 = edge_kernel(sa_pk, sp_pk,
                       src.reshape(rows, 128), dst.reshape(rows, 128))
    out = out2.reshape(e_pad)
    return out[:e] if e_pad != e else out


# speedup vs baseline: 1.0014x; 1.0014x over previous
import functools

import jax
import jax.numpy as jnp
from jax import lax
from jax.experimental import pallas as pl
from jax.experimental.pallas import tpu as pltpu
from jax.experimental.pallas import tpu_sc as plsc

_NC = 2
_NS = 16
_NW = _NC * _NS
_L = 16


def _rne_bf16_bits(x):
    b = lax.bitcast_convert_type(x, jnp.int32)
    r = b + 0x7FFF + (lax.shift_right_logical(b, 16) & 1)
    return lax.shift_right_logical(r, 16)


def _encode_body(ax_ref, px_ref, pae_ref, pao_ref, ppe_ref, ppo_ref, c_ref,
                 oa_ref, op_ref):
    ae = jnp.dot(ax_ref[...], pae_ref[...], preferred_element_type=jnp.float32)
    ao = jnp.dot(ax_ref[...], pao_ref[...], preferred_element_type=jnp.float32)
    oa_ref[...] = _rne_bf16_bits(ae) | (_rne_bf16_bits(ao) << 16)
    cc = c_ref[0]
    pe = jnp.dot(px_ref[...], ppe_ref[...], preferred_element_type=jnp.float32)
    po = jnp.dot(px_ref[...], ppo_ref[...], preferred_element_type=jnp.float32)
    op_ref[...] = _rne_bf16_bits(pe + cc) | (_rne_bf16_bits(po + cc) << 16)


def _encode(ax2, px2, pae, pao, ppe, ppo, const):
    ra = ax2.shape[0]
    rp = px2.shape[0]
    return pl.pallas_call(
        _encode_body,
        out_shape=[
            jax.ShapeDtypeStruct((ra, 128), jnp.int32),
            jax.ShapeDtypeStruct((rp, 128), jnp.int32),
        ],
        in_specs=[
            pl.BlockSpec(memory_space=pltpu.VMEM),
            pl.BlockSpec(memory_space=pltpu.VMEM),
            pl.BlockSpec(memory_space=pltpu.VMEM),
            pl.BlockSpec(memory_space=pltpu.VMEM),
            pl.BlockSpec(memory_space=pltpu.VMEM),
            pl.BlockSpec(memory_space=pltpu.VMEM),
            pl.BlockSpec(memory_space=pltpu.SMEM),
        ],
    )(ax2, px2, pae, pao, ppe, ppo, const)


_CROWS = 32
_KE = _CROWS * 128


def _make_edge_kernel(rows_total, ta_rows, tp_rows):
    rows8 = rows_total // 8
    base8 = rows8 // _NW
    extra8 = rows8 - base8 * _NW
    chunks = -(-(base8 + 1) * 8 // _CROWS)
    steps = _KE // _L
    mesh = plsc.VectorSubcoreMesh(
        core_axis_name="c", subcore_axis_name="s",
        num_cores=_NC, num_subcores=_NS)

    @functools.partial(
        pl.kernel,
        out_type=jax.ShapeDtypeStruct((rows_total, 128), jnp.float32),
        mesh=mesh,
        compiler_params=pltpu.CompilerParams(
            needs_layout_passes=False, use_tc_tiling_on_sc=True),
        scratch_types=[
            pltpu.VMEM((ta_rows, 128), jnp.int32),
            pltpu.VMEM((tp_rows, 128), jnp.int32),
            pltpu.VMEM((_CROWS, 128), jnp.int32),
            pltpu.VMEM((_CROWS, 128), jnp.int32),
            pltpu.VMEM((_CROWS, 128), jnp.float32),
        ],
    )
    def edge_kernel(sa_hbm, sp_hbm, src_hbm, dst_hbm, out_hbm,
                    sa_v, sp_v, src_v, dst_v, out_v):
        wid = lax.axis_index("s") * _NC + lax.axis_index("c")
        row_lo = (wid * base8 + jnp.minimum(wid, extra8)) * 8
        n_rows = (base8 + jnp.where(wid < extra8, 1, 0)) * 8
        last_off = row_lo + n_rows - _CROWS
        pltpu.sync_copy(sa_hbm, sa_v)
        pltpu.sync_copy(sp_hbm, sp_v)

        def chunk(c, carry):
            off = pl.multiple_of(
                jnp.minimum(row_lo + c * _CROWS, last_off), 8)
            pltpu.sync_copy(src_hbm.at[pl.ds(off, _CROWS), :], src_v)
            pltpu.sync_copy(dst_hbm.at[pl.ds(off, _CROWS), :], dst_v)

            def inner(i, carry2):
                r = i >> 3
                cg = (i & 7) * _L
                sl = pl.ds(cg, _L)
                s = src_v[r, sl]
                d = dst_v[r, sl]
                ws = plsc.load_gather(sa_v, [s >> 8, (s >> 1) & 127])
                wd = plsc.load_gather(sp_v, [d >> 8, (d >> 1) & 127])
                fs = plsc.bitcast(
                    lax.shift_right_logical(ws, (s & 1) << 4) << 16,
                    jnp.float32)
                fd = plsc.bitcast(
                    lax.shift_right_logical(wd, (d & 1) << 4) << 16,
                    jnp.float32)
                out_v[r, sl] = fs + fd
                return carry2

            lax.fori_loop(0, steps, inner, 0, unroll=2)
            pltpu.sync_copy(out_v, out_hbm.at[pl.ds(off, _CROWS), :])
            return carry

        lax.fori_loop(0, chunks, chunk, 0)

    return edge_kernel


def _pattern(v, d):
    l = jnp.arange(256 * d)
    c = jnp.arange(128)
    vt = jnp.tile(v, 256)
    node = l[:, None] // d
    pe = jnp.where(node == 2 * c[None, :], vt[:, None], 0.0)
    po = jnp.where(node == 2 * c[None, :] + 1, vt[:, None], 0.0)
    return pe.astype(jnp.float32), po.astype(jnp.float32)


def _pad_nodes(x, n_pad):
    n = x.shape[0]
    if n_pad != n:
        x = jnp.concatenate(
            [x, jnp.zeros((n_pad - n, x.shape[1]), x.dtype)], axis=0)
    return x


def kernel(author_x, paper_x, src_index, dst_index,
           W_author, b_author, W_paper, b_paper, W_scorer, b_scorer):
    d = author_x.shape[1]
    na = author_x.shape[0]
    npp = paper_x.shape[0]
    e = src_index.shape[0]
    npn = 1024 // d

    w1 = W_scorer[0, :d]
    w2 = W_scorer[0, d:]
    v_a = W_author.T @ w1
    v_p = W_paper.T @ w2
    const = (b_scorer[0] + b_author @ w1 + b_paper @ w2).reshape(1)

    pae, pao = _pattern(v_a, d)
    ppe, ppo = _pattern(v_p, d)

    na_pad = -(-na // npn) * npn
    np_pad = -(-npp // npn) * npn
    ax2 = _pad_nodes(author_x, na_pad).reshape(na_pad // npn, npn * d)
    px2 = _pad_nodes(paper_x, np_pad).reshape(np_pad // npn, npn * d)

    sa_pk, sp_pk = _encode(ax2, px2, pae, pao, ppe, ppo, const)

    src = src_index.astype(jnp.int32)
    dst = dst_index.astype(jnp.int32)
    min_e = _KE * _NW
    e_pad = -(-max(e, min_e) // 1024) * 1024
    if e_pad != e:
        pad = e_pad - e
        src = jnp.concatenate([src, jnp.zeros((pad,), jnp.int32)])
        dst = jnp.concatenate([dst, jnp.zeros((pad,), jnp.int32)])
    rows = e_pad // 128

    edge_kernel = _make_edge_kernel(rows, sa_pk.shape[0], sp_pk.shape[0])
    out2 = edge_kernel(sa_pk, sp_pk,
                       src.reshape(rows, 128), dst.reshape(rows, 128))
    out = out2.reshape(e_pad)
    return out[:e] if e_pad != e else out


# speedup vs baseline: 1.4711x; 1.4691x over previous
import functools

import jax
import jax.numpy as jnp
from jax import lax
from jax.experimental import pallas as pl
from jax.experimental.pallas import tpu as pltpu
from jax.experimental.pallas import tpu_sc as plsc

_NC = 2
_NS = 16
_NW = _NC * _NS
_L = 16

_GB = 4096


def _rne_bf16_bits(x):
    b = lax.bitcast_convert_type(x, jnp.int32)
    r = b + 0x7FFF + (lax.shift_right_logical(b, 16) & 1)
    return lax.shift_right_logical(r, 16)


def _pack8(s):
    rows = [
        _rne_bf16_bits(s[:, 512 * j:512 * j + 256])
        | (_rne_bf16_bits(s[:, 512 * j + 256:512 * j + 512]) << 16)
        for j in range(8)
    ]
    return jnp.concatenate(rows, axis=0)


def _encode_body(ax_ref, px_ref, va_ref, vp_ref, c_ref, oa_ref, op_ref):
    asum = jnp.sum(ax_ref[...] * va_ref[...], axis=0, keepdims=True)
    oa_ref[...] = _pack8(asum)
    psum = jnp.sum(px_ref[...] * vp_ref[...], axis=0, keepdims=True) + c_ref[0]
    op_ref[...] = _pack8(psum)


def _encode(axt, pxt, va, vp, const):
    d, na = axt.shape
    npp = pxt.shape[1]
    grid = -(-max(na, npp) // _GB)
    nba = -(-na // _GB) - 1
    nbp = -(-npp // _GB) - 1

    return pl.pallas_call(
        _encode_body,
        grid=(grid,),
        out_shape=[
            jax.ShapeDtypeStruct((8 * grid, 256), jnp.int32),
            jax.ShapeDtypeStruct((8 * grid, 256), jnp.int32),
        ],
        in_specs=[
            pl.BlockSpec((d, _GB), lambda g: (0, jnp.minimum(g, nba))),
            pl.BlockSpec((d, _GB), lambda g: (0, jnp.minimum(g, nbp))),
            pl.BlockSpec((d, 1), lambda g: (0, 0)),
            pl.BlockSpec((d, 1), lambda g: (0, 0)),
            pl.BlockSpec(memory_space=pltpu.SMEM),
        ],
        out_specs=[
            pl.BlockSpec((8, 256), lambda g: (g, 0)),
            pl.BlockSpec((8, 256), lambda g: (g, 0)),
        ],
    )(axt, pxt, va, vp, const)


_CROWS = 32
_KE = _CROWS * 128


def _make_edge_kernel(rows_total, ta_rows, tp_rows):
    rows8 = rows_total // 8
    base8 = rows8 // _NW
    extra8 = rows8 - base8 * _NW
    chunks = -(-(base8 + 1) * 8 // _CROWS)
    steps = _KE // _L
    mesh = plsc.VectorSubcoreMesh(
        core_axis_name="c", subcore_axis_name="s",
        num_cores=_NC, num_subcores=_NS)

    @functools.partial(
        pl.kernel,
        out_type=jax.ShapeDtypeStruct((rows_total, 128), jnp.float32),
        mesh=mesh,
        compiler_params=pltpu.CompilerParams(
            needs_layout_passes=False, use_tc_tiling_on_sc=True),
        scratch_types=[
            pltpu.VMEM((ta_rows, 256), jnp.int32),
            pltpu.VMEM((tp_rows, 256), jnp.int32),
            pltpu.VMEM((_CROWS, 128), jnp.int32),
            pltpu.VMEM((_CROWS, 128), jnp.int32),
            pltpu.VMEM((_CROWS, 128), jnp.float32),
        ],
    )
    def edge_kernel(sa_hbm, sp_hbm, src_hbm, dst_hbm, out_hbm,
                    sa_v, sp_v, src_v, dst_v, out_v):
        wid = lax.axis_index("s") * _NC + lax.axis_index("c")
        row_lo = (wid * base8 + jnp.minimum(wid, extra8)) * 8
        n_rows = (base8 + jnp.where(wid < extra8, 1, 0)) * 8
        last_off = row_lo + n_rows - _CROWS
        pltpu.sync_copy(sa_hbm, sa_v)
        pltpu.sync_copy(sp_hbm, sp_v)

        def chunk(c, carry):
            off = pl.multiple_of(
                jnp.minimum(row_lo + c * _CROWS, last_off), 8)
            pltpu.sync_copy(src_hbm.at[pl.ds(off, _CROWS), :], src_v)
            pltpu.sync_copy(dst_hbm.at[pl.ds(off, _CROWS), :], dst_v)

            def inner(i, carry2):
                r = i >> 3
                cg = (i & 7) * _L
                sl = pl.ds(cg, _L)
                s = src_v[r, sl]
                d = dst_v[r, sl]
                ws = plsc.load_gather(sa_v, [s >> 9, s & 255])
                wd = plsc.load_gather(sp_v, [d >> 9, d & 255])
                fs = plsc.bitcast(
                    lax.shift_right_logical(ws, (s & 256) >> 4) << 16,
                    jnp.float32)
                fd = plsc.bitcast(
                    lax.shift_right_logical(wd, (d & 256) >> 4) << 16,
                    jnp.float32)
                out_v[r, sl] = fs + fd
                return carry2

            lax.fori_loop(0, steps, inner, 0, unroll=2)
            pltpu.sync_copy(out_v, out_hbm.at[pl.ds(off, _CROWS), :])
            return carry

        lax.fori_loop(0, chunks, chunk, 0)

    return edge_kernel


def kernel(author_x, paper_x, src_index, dst_index,
           W_author, b_author, W_paper, b_paper, W_scorer, b_scorer):
    d = author_x.shape[1]
    e = src_index.shape[0]

    w1 = W_scorer[0, :d]
    w2 = W_scorer[0, d:]
    v_a = (W_author.T @ w1).reshape(d, 1)
    v_p = (W_paper.T @ w2).reshape(d, 1)
    const = (b_scorer[0] + b_author @ w1 + b_paper @ w2).reshape(1)

    sa_pk, sp_pk = _encode(author_x.T, paper_x.T, v_a, v_p, const)

    src = src_index.astype(jnp.int32)
    dst = dst_index.astype(jnp.int32)
    min_e = _KE * _NW
    e_pad = -(-max(e, min_e) // 1024) * 1024
    if e_pad != e:
        pad = e_pad - e
        src = jnp.concatenate([src, jnp.zeros((pad,), jnp.int32)])
        dst = jnp.concatenate([dst, jnp.zeros((pad,), jnp.int32)])
    rows = e_pad // 128

    edge_kernel = _make_edge_kernel(rows, sa_pk.shape[0], sp_pk.shape[0])
    out2 = edge_kernel(sa_pk, sp_pk,
                       src.reshape(rows, 128), dst.reshape(rows, 128))
    out = out2.reshape(e_pad)
    return out[:e] if e_pad != e else out


# speedup vs baseline: 3.3153x; 2.2536x over previous
import functools

import jax
import jax.numpy as jnp
from jax import lax
from jax.experimental import pallas as pl
from jax.experimental.pallas import tpu as pltpu
from jax.experimental.pallas import tpu_sc as plsc

_NC = 2
_NS = 16
_NW = _NC * _NS
_L = 16

_GB = 4096


def _rne_bf16_bits(x):
    b = lax.bitcast_convert_type(x, jnp.int32)
    r = b + 0x7FFF + (lax.shift_right_logical(b, 16) & 1)
    return lax.shift_right_logical(r, 16)


def _pack8(s):
    rows = [
        _rne_bf16_bits(s[:, 512 * j:512 * j + 256])
        | (_rne_bf16_bits(s[:, 512 * j + 256:512 * j + 512]) << 16)
        for j in range(8)
    ]
    return jnp.concatenate(rows, axis=0)


def _encode_body(ax_ref, px_ref, va_ref, vp_ref, c_ref, oa_ref, op_ref):
    asum = jnp.sum(ax_ref[...] * va_ref[...], axis=0, keepdims=True)
    oa_ref[...] = _pack8(asum)
    psum = jnp.sum(px_ref[...] * vp_ref[...], axis=0, keepdims=True) + c_ref[0]
    op_ref[...] = _pack8(psum)


def _encode(axt, pxt, va, vp, const):
    d, na = axt.shape
    npp = pxt.shape[1]
    grid = -(-max(na, npp) // _GB)
    nba = -(-na // _GB) - 1
    nbp = -(-npp // _GB) - 1

    return pl.pallas_call(
        _encode_body,
        grid=(grid,),
        out_shape=[
            jax.ShapeDtypeStruct((8 * grid, 256), jnp.int32),
            jax.ShapeDtypeStruct((8 * grid, 256), jnp.int32),
        ],
        in_specs=[
            pl.BlockSpec((d, _GB), lambda g: (0, jnp.minimum(g, nba))),
            pl.BlockSpec((d, _GB), lambda g: (0, jnp.minimum(g, nbp))),
            pl.BlockSpec((d, 1), lambda g: (0, 0)),
            pl.BlockSpec((d, 1), lambda g: (0, 0)),
            pl.BlockSpec(memory_space=pltpu.SMEM),
        ],
        out_specs=[
            pl.BlockSpec((8, 256), lambda g: (g, 0)),
            pl.BlockSpec((8, 256), lambda g: (g, 0)),
        ],
    )(axt, pxt, va, vp, const)


_CROWS = 32
_KE = _CROWS * 128


def _make_edge_kernel(rows_total, ta_rows, tp_rows):
    rows8 = rows_total // 8
    base8 = rows8 // _NW
    extra8 = rows8 - base8 * _NW
    chunks = -(-(base8 + 1) * 8 // _CROWS)
    steps = _KE // _L
    mesh = plsc.VectorSubcoreMesh(
        core_axis_name="c", subcore_axis_name="s",
        num_cores=_NC, num_subcores=_NS)

    @functools.partial(
        pl.kernel,
        out_type=jax.ShapeDtypeStruct((rows_total, 128), jnp.float32),
        mesh=mesh,
        compiler_params=pltpu.CompilerParams(
            needs_layout_passes=False, use_tc_tiling_on_sc=True),
        scratch_types=[
            pltpu.VMEM((ta_rows, 256), jnp.int32),
            pltpu.VMEM((tp_rows, 256), jnp.int32),
            pltpu.VMEM((2, _CROWS, 128), jnp.int32),
            pltpu.VMEM((2, _CROWS, 128), jnp.int32),
            pltpu.VMEM((2, _CROWS, 128), jnp.float32),
            pltpu.SemaphoreType.DMA,
            pltpu.SemaphoreType.DMA,
            pltpu.SemaphoreType.DMA,
            pltpu.SemaphoreType.DMA,
        ],
    )
    def edge_kernel(sa_hbm, sp_hbm, src_hbm, dst_hbm, out_hbm,
                    sa_v, sp_v, src_v, dst_v, out_v,
                    s_in0, s_in1, s_out0, s_out1):
        s_in = (s_in0, s_in1)
        s_out = (s_out0, s_out1)
        wid = lax.axis_index("s") * _NC + lax.axis_index("c")
        row_lo = (wid * base8 + jnp.minimum(wid, extra8)) * 8
        n_rows = (base8 + jnp.where(wid < extra8, 1, 0)) * 8
        last_off = row_lo + n_rows - _CROWS

        def off_of(c):
            return pl.multiple_of(
                jnp.minimum(row_lo + c * _CROWS, last_off), 8)

        def start_in(c, b):
            off = off_of(c)
            pltpu.async_copy(src_hbm.at[pl.ds(off, _CROWS), :],
                             src_v.at[b], s_in[b])
            pltpu.async_copy(dst_hbm.at[pl.ds(off, _CROWS), :],
                             dst_v.at[b], s_in[b])

        def wait_in(c, b):
            off = off_of(c)
            pltpu.make_async_copy(src_hbm.at[pl.ds(off, _CROWS), :],
                                  src_v.at[b], s_in[b]).wait()
            pltpu.make_async_copy(dst_hbm.at[pl.ds(off, _CROWS), :],
                                  dst_v.at[b], s_in[b]).wait()

        def wait_out(c, b):
            off = off_of(c)
            pltpu.make_async_copy(out_v.at[b],
                                  out_hbm.at[pl.ds(off, _CROWS), :],
                                  s_out[b]).wait()

        pltpu.sync_copy(sa_hbm, sa_v)
        pltpu.sync_copy(sp_hbm, sp_v)
        start_in(0, 0)
        start_in(1, 1)

        def pair(cc, carry):
            for b in (0, 1):
                c = cc * 2 + b

                @pl.when(c < chunks)
                def _():
                    wait_in(c, b)

                    @pl.when(c >= 2)
                    def _():
                        wait_out(c - 2, b)

                    def inner(r, carry2):
                        for j in range(8):
                            sl = pl.ds(j * _L, _L)
                            s = src_v[b, r, sl]
                            d = dst_v[b, r, sl]
                            ws = plsc.load_gather(sa_v, [s >> 9, s & 255])
                            wd = plsc.load_gather(sp_v, [d >> 9, d & 255])
                            fs = plsc.bitcast(
                                lax.shift_right_logical(
                                    ws, (s & 256) >> 4) << 16, jnp.float32)
                            fd = plsc.bitcast(
                                lax.shift_right_logical(
                                    wd, (d & 256) >> 4) << 16, jnp.float32)
                            out_v[b, r, sl] = fs + fd
                        return carry2

                    lax.fori_loop(0, _CROWS, inner, 0)
                    pltpu.async_copy(out_v.at[b],
                                     out_hbm.at[pl.ds(off_of(c), _CROWS), :],
                                     s_out[b])

                    @pl.when(c + 2 < chunks)
                    def _():
                        start_in(c + 2, b)
            return carry

        lax.fori_loop(0, (chunks + 1) // 2, pair, 0)
        wait_out(chunks - 1, (chunks - 1) % 2)
        wait_out(chunks - 2, (chunks - 2) % 2)

    return edge_kernel


def kernel(author_x, paper_x, src_index, dst_index,
           W_author, b_author, W_paper, b_paper, W_scorer, b_scorer):
    d = author_x.shape[1]
    e = src_index.shape[0]

    w1 = W_scorer[0, :d]
    w2 = W_scorer[0, d:]
    v_a = (W_author.T @ w1).reshape(d, 1)
    v_p = (W_paper.T @ w2).reshape(d, 1)
    const = (b_scorer[0] + b_author @ w1 + b_paper @ w2).reshape(1)

    sa_pk, sp_pk = _encode(author_x.T, paper_x.T, v_a, v_p, const)

    src = src_index.astype(jnp.int32)
    dst = dst_index.astype(jnp.int32)
    min_e = _KE * _NW
    e_pad = -(-max(e, min_e) // 1024) * 1024
    if e_pad != e:
        pad = e_pad - e
        src = jnp.concatenate([src, jnp.zeros((pad,), jnp.int32)])
        dst = jnp.concatenate([dst, jnp.zeros((pad,), jnp.int32)])
    rows = e_pad // 128

    edge_kernel = _make_edge_kernel(rows, sa_pk.shape[0], sp_pk.shape[0])
    out2 = edge_kernel(sa_pk, sp_pk,
                       src.reshape(rows, 128), dst.reshape(rows, 128))
    out = out2.reshape(e_pad)
    return out[:e] if e_pad != e else out


# speedup vs baseline: 3.7533x; 1.1321x over previous
import functools

import jax
import jax.numpy as jnp
from jax import lax
from jax.experimental import pallas as pl
from jax.experimental.pallas import tpu as pltpu
from jax.experimental.pallas import tpu_sc as plsc

_NC = 2
_NS = 16
_NW = _NC * _NS
_L = 16

_GB = 4096


def _rne_bf16_bits(x):
    b = lax.bitcast_convert_type(x, jnp.int32)
    r = b + 0x7FFF + (lax.shift_right_logical(b, 16) & 1)
    return lax.shift_right_logical(r, 16)


def _pack8(s):
    rows = [
        _rne_bf16_bits(s[:, 512 * j:512 * j + 256])
        | (_rne_bf16_bits(s[:, 512 * j + 256:512 * j + 512]) << 16)
        for j in range(8)
    ]
    return jnp.concatenate(rows, axis=0)


def _encode_body(ax_ref, px_ref, va_ref, vp_ref, c_ref, oa_ref, op_ref):
    asum = jnp.sum(ax_ref[...] * va_ref[...], axis=0, keepdims=True)
    oa_ref[...] = _pack8(asum)
    psum = jnp.sum(px_ref[...] * vp_ref[...], axis=0, keepdims=True) + c_ref[0]
    op_ref[...] = _pack8(psum)


def _encode(axt, pxt, va, vp, const):
    d, na = axt.shape
    npp = pxt.shape[1]
    grid = -(-max(na, npp) // _GB)
    nba = -(-na // _GB) - 1
    nbp = -(-npp // _GB) - 1

    return pl.pallas_call(
        _encode_body,
        grid=(grid,),
        out_shape=[
            jax.ShapeDtypeStruct((8 * grid, 256), jnp.int32),
            jax.ShapeDtypeStruct((8 * grid, 256), jnp.int32),
        ],
        in_specs=[
            pl.BlockSpec((d, _GB), lambda g: (0, jnp.minimum(g, nba))),
            pl.BlockSpec((d, _GB), lambda g: (0, jnp.minimum(g, nbp))),
            pl.BlockSpec((d, 1), lambda g: (0, 0)),
            pl.BlockSpec((d, 1), lambda g: (0, 0)),
            pl.BlockSpec(memory_space=pltpu.SMEM),
        ],
        out_specs=[
            pl.BlockSpec((8, 256), lambda g: (g, 0)),
            pl.BlockSpec((8, 256), lambda g: (g, 0)),
        ],
    )(axt, pxt, va, vp, const)


_CROWS = 32
_KE = _CROWS * 128


def _make_edge_kernel(rows_total, ta_rows, tp_rows):
    rows8 = rows_total // 8
    base8 = rows8 // _NW
    extra8 = rows8 - base8 * _NW
    chunks = -(-(base8 + 1) * 8 // _CROWS)
    steps = _KE // _L
    mesh = plsc.VectorSubcoreMesh(
        core_axis_name="c", subcore_axis_name="s",
        num_cores=_NC, num_subcores=_NS)

    @functools.partial(
        pl.kernel,
        out_type=jax.ShapeDtypeStruct((rows_total, 128), jnp.float32),
        mesh=mesh,
        compiler_params=pltpu.CompilerParams(
            needs_layout_passes=False, use_tc_tiling_on_sc=True),
        scratch_types=[
            pltpu.VMEM((ta_rows, 256), jnp.int32),
            pltpu.VMEM((tp_rows, 256), jnp.int32),
            pltpu.VMEM((2, _CROWS, 128), jnp.int32),
            pltpu.VMEM((2, _CROWS, 128), jnp.int32),
            pltpu.VMEM((2, _CROWS, 128), jnp.float32),
            pltpu.SemaphoreType.DMA,
            pltpu.SemaphoreType.DMA,
            pltpu.SemaphoreType.DMA,
            pltpu.SemaphoreType.DMA,
        ],
    )
    def edge_kernel(sa_hbm, sp_hbm, src_hbm, dst_hbm, out_hbm,
                    sa_v, sp_v, src_v, dst_v, out_v,
                    s_in0, s_in1, s_out0, s_out1):
        s_in = (s_in0, s_in1)
        s_out = (s_out0, s_out1)
        wid = lax.axis_index("s") * _NC + lax.axis_index("c")
        row_lo = (wid * base8 + jnp.minimum(wid, extra8)) * 8
        n_rows = (base8 + jnp.where(wid < extra8, 1, 0)) * 8
        last_off = row_lo + n_rows - _CROWS

        def off_of(c):
            return pl.multiple_of(
                jnp.minimum(row_lo + c * _CROWS, last_off), 8)

        def start_in(c, b):
            off = off_of(c)
            pltpu.async_copy(src_hbm.at[pl.ds(off, _CROWS), :],
                             src_v.at[b], s_in[b])
            pltpu.async_copy(dst_hbm.at[pl.ds(off, _CROWS), :],
                             dst_v.at[b], s_in[b])

        def wait_in(c, b):
            off = off_of(c)
            pltpu.make_async_copy(src_hbm.at[pl.ds(off, _CROWS), :],
                                  src_v.at[b], s_in[b]).wait()
            pltpu.make_async_copy(dst_hbm.at[pl.ds(off, _CROWS), :],
                                  dst_v.at[b], s_in[b]).wait()

        def wait_out(c, b):
            off = off_of(c)
            pltpu.make_async_copy(out_v.at[b],
                                  out_hbm.at[pl.ds(off, _CROWS), :],
                                  s_out[b]).wait()

        pltpu.sync_copy(sa_hbm, sa_v)
        pltpu.sync_copy(sp_hbm, sp_v)
        start_in(0, 0)
        start_in(1, 1)

        def pair(cc, carry):
            for b in (0, 1):
                c = cc * 2 + b

                @pl.when(c < chunks)
                def _():
                    wait_in(c, b)

                    @pl.when(c >= 2)
                    def _():
                        wait_out(c - 2, b)

                    @plsc.parallel_loop(0, _CROWS, unroll=2)
                    def _(r):
                        for j in range(8):
                            sl = pl.ds(j * _L, _L)
                            s = src_v[b, r, sl]
                            d = dst_v[b, r, sl]
                            ws = plsc.load_gather(sa_v, [s >> 9, s & 255])
                            wd = plsc.load_gather(sp_v, [d >> 9, d & 255])
                            fs = plsc.bitcast(
                                lax.shift_right_logical(
                                    ws, (s & 256) >> 4) << 16, jnp.float32)
                            fd = plsc.bitcast(
                                lax.shift_right_logical(
                                    wd, (d & 256) >> 4) << 16, jnp.float32)
                            out_v[b, r, sl] = fs + fd
                    pltpu.async_copy(out_v.at[b],
                                     out_hbm.at[pl.ds(off_of(c), _CROWS), :],
                                     s_out[b])

                    @pl.when(c + 2 < chunks)
                    def _():
                        start_in(c + 2, b)
            return carry

        lax.fori_loop(0, (chunks + 1) // 2, pair, 0)
        wait_out(chunks - 1, (chunks - 1) % 2)
        wait_out(chunks - 2, (chunks - 2) % 2)

    return edge_kernel


def kernel(author_x, paper_x, src_index, dst_index,
           W_author, b_author, W_paper, b_paper, W_scorer, b_scorer):
    d = author_x.shape[1]
    e = src_index.shape[0]

    w1 = W_scorer[0, :d]
    w2 = W_scorer[0, d:]
    v_a = (W_author.T @ w1).reshape(d, 1)
    v_p = (W_paper.T @ w2).reshape(d, 1)
    const = (b_scorer[0] + b_author @ w1 + b_paper @ w2).reshape(1)

    sa_pk, sp_pk = _encode(author_x.T, paper_x.T, v_a, v_p, const)

    src = src_index.astype(jnp.int32)
    dst = dst_index.astype(jnp.int32)
    min_e = _KE * _NW
    e_pad = -(-max(e, min_e) // 1024) * 1024
    if e_pad != e:
        pad = e_pad - e
        src = jnp.concatenate([src, jnp.zeros((pad,), jnp.int32)])
        dst = jnp.concatenate([dst, jnp.zeros((pad,), jnp.int32)])
    rows = e_pad // 128

    edge_kernel = _make_edge_kernel(rows, sa_pk.shape[0], sp_pk.shape[0])
    out2 = edge_kernel(sa_pk, sp_pk,
                       src.reshape(rows, 128), dst.reshape(rows, 128))
    out = out2.reshape(e_pad)
    return out[:e] if e_pad != e else out
